# baseline (speedup 1.0000x reference)
import functools

import jax
import jax.numpy as jnp
from jax import lax
from jax.experimental import pallas as pl
from jax.experimental.pallas import tpu as pltpu
from jax.experimental.pallas import tpu_sc as plsc

B = 4
S = 2048
IN_DIM = 1024
HID = 2048
OUT = 2048
SEGS = 2
SEG_LEN = 256

N_EMB = B * SEGS * SEG_LEN
N_TOK = B * S
CAT = SEGS * (SEG_LEN + 1)
N_SEQ = B * (CAT + S)

NC = 2
NS = 16
NW = NC * NS

_sc_mesh = plsc.VectorSubcoreMesh(core_axis_name="c", subcore_axis_name="s")


def _make_sc_gather(n_rows, dim, chunk):
    per_w = n_rows // NW
    n_chunks = per_w // chunk
    assert per_w % chunk == 0 and n_rows % NW == 0 and chunk % 8 == 0

    @functools.partial(
        pl.kernel,
        out_type=jax.ShapeDtypeStruct((n_rows, dim), jnp.float32),
        mesh=_sc_mesh,
        scratch_types=[
            pltpu.VMEM((per_w,), jnp.int32),
            pltpu.VMEM((chunk, dim), jnp.float32),
            pltpu.VMEM((chunk, dim), jnp.float32),
            pltpu.SemaphoreType.DMA,
            pltpu.SemaphoreType.DMA,
            pltpu.SemaphoreType.DMA,
            pltpu.SemaphoreType.DMA,
        ],
    )
    def gather(table_hbm, idx_hbm, out_hbm, idx_v, rows0, rows1,
               sg0, sg1, sw0, sw1):
        wid = lax.axis_index("s") * NC + lax.axis_index("c")
        base = wid * per_w
        pltpu.sync_copy(idx_hbm.at[pl.ds(base, per_w)], idx_v)
        bufs = [(rows0, sg0, sw0), (rows1, sg1, sw1)]
        g = [None] * n_chunks
        w = [None] * n_chunks
        for c in range(n_chunks + 1):
            if c < n_chunks:
                rows, sg, sw = bufs[c % 2]
                if c >= 2:
                    w[c - 2].wait()
                g[c] = pltpu.async_copy(
                    table_hbm.at[idx_v.at[pl.ds(c * chunk, chunk)]], rows, sg)
            if c >= 1:
                rows_p, _, sw_p = bufs[(c - 1) % 2]
                g[c - 1].wait()
                w[c - 1] = pltpu.async_copy(
                    rows_p, out_hbm.at[pl.ds(base + (c - 1) * chunk, chunk)],
                    sw_p)
        for c in range(max(0, n_chunks - 2), n_chunks):
            w[c].wait()

    return gather


_gather_emb = _make_sc_gather(N_EMB, IN_DIM, 16)
_gather_tok = _make_sc_gather(N_TOK, OUT, 16)


_AS_CHUNK = 16


def _copy_rows(src, dst, src_off, dst_off, n_rows, bufs, start_c=0):
    n_chunks = n_rows // _AS_CHUNK
    cw = _AS_CHUNK * OUT
    g = [None] * n_chunks
    w = [None] * n_chunks
    nb = len(bufs)
    for c in range(n_chunks + 1):
        if c < n_chunks:
            rows, sg, sw = bufs[(start_c + c) % nb]
            if c >= nb:
                w[c - nb].wait()
            g[c] = pltpu.async_copy(
                src.at[pl.ds((src_off + c * _AS_CHUNK) * OUT, cw)], rows, sg)
        if c >= 1:
            rows_p, _, sw_p = bufs[(start_c + c - 1) % nb]
            g[c - 1].wait()
            w[c - 1] = pltpu.async_copy(
                rows_p,
                dst.at[pl.ds((dst_off + (c - 1) * _AS_CHUNK) * OUT, cw)],
                sw_p)
    for c in range(max(0, n_chunks - nb), n_chunks):
        w[c].wait()
    return start_c + n_chunks


@functools.partial(
    pl.kernel,
    out_type=jax.ShapeDtypeStruct((N_SEQ * OUT,), jnp.float32),
    mesh=_sc_mesh,
    scratch_types=[
        pltpu.VMEM((_AS_CHUNK * OUT,), jnp.float32),
        pltpu.VMEM((_AS_CHUNK * OUT,), jnp.float32),
        pltpu.VMEM((OUT,), jnp.float32),
        pltpu.SemaphoreType.DMA,
        pltpu.SemaphoreType.DMA,
        pltpu.SemaphoreType.DMA,
        pltpu.SemaphoreType.DMA,
    ],
)
def _assemble(h_hbm, tok_hbm, out_hbm, rows0, rows1, sp_v,
              sg0, sg1, sw0, sw1):
    wid = lax.axis_index("s") * NC + lax.axis_index("c")
    bufs = [(rows0, sg0, sw0), (rows1, sg1, sw1)]
    t0 = wid * 256
    tb = wid // 8
    nxt = _copy_rows(tok_hbm, out_hbm, t0,
                     (CAT + S) * tb + CAT + (t0 - S * tb), 256, bufs)
    g0 = wid * 64
    gb = wid // 8
    gs = (wid % 8) // 4
    gp = (wid % 4) * 64
    _copy_rows(h_hbm, out_hbm, g0,
               (CAT + S) * gb + (SEG_LEN + 1) * gs + gp, 64, bufs,
               start_c=nxt)
    @pl.when(wid < 8)
    def _():
        pltpu.sync_copy(h_hbm.at[pl.ds(N_EMB * OUT, OUT)], sp_v)
        sb = wid // 2
        ss = wid % 2
        pltpu.sync_copy(
            sp_v,
            out_hbm.at[pl.ds(((CAT + S) * sb + (SEG_LEN + 1) * ss + SEG_LEN)
                             * OUT, OUT)])


_CAST_G = 16


def _cast_body(a_ref, b_ref, ao_ref, bo_ref):
    ao_ref[...] = a_ref[...].astype(jnp.bfloat16)
    bo_ref[...] = b_ref[...].astype(jnp.bfloat16)


def _cast_weights(wf1, wf2):
    shapes = [wf1.shape, wf2.shape]
    blocks = [(s[0] // _CAST_G, s[1]) for s in shapes]
    return pl.pallas_call(
        _cast_body,
        grid=(_CAST_G,),
        in_specs=[pl.BlockSpec(blk, lambda i: (i, 0)) for blk in blocks],
        out_specs=[pl.BlockSpec(blk, lambda i: (i, 0)) for blk in blocks],
        out_shape=[jax.ShapeDtypeStruct(s, jnp.bfloat16) for s in shapes],
    )(wf1, wf2)


_BR_ROWS = N_EMB + 128
_BR_TI = 128
_BR_NB = _BR_ROWS // _BR_TI


def _bridge_body(e_ref, sp_ref, w1_ref, b1_ref, w2_ref, b2_ref, o_ref):
    i = pl.program_id(0)
    e = e_ref[...]
    sp = jnp.broadcast_to(sp_ref[...], e.shape)
    e = jnp.where(i == _BR_NB - 1, sp, e)
    h = jnp.dot(e, w1_ref[...], preferred_element_type=jnp.float32)
    h = jax.nn.gelu(h + b1_ref[...])
    o_ref[...] = (
        jnp.dot(h, w2_ref[...], preferred_element_type=jnp.float32)
        + b2_ref[...]
    )


def _bridge_call(e_rows, special_tok, w1, b1, w2, b2):
    nb = _BR_NB
    return pl.pallas_call(
        _bridge_body,
        grid=(nb,),
        in_specs=[
            pl.BlockSpec((_BR_TI, IN_DIM),
                         lambda i: (jnp.minimum(i, nb - 2), 0)),
            pl.BlockSpec((1, IN_DIM), lambda i: (0, 0)),
            pl.BlockSpec((IN_DIM, HID), lambda i: (0, 0)),
            pl.BlockSpec((1, HID), lambda i: (0, 0)),
            pl.BlockSpec((HID, OUT), lambda i: (0, 0)),
            pl.BlockSpec((1, OUT), lambda i: (0, 0)),
        ],
        out_specs=pl.BlockSpec((_BR_TI, OUT), lambda i: (i, 0)),
        out_shape=jax.ShapeDtypeStruct((_BR_ROWS, OUT), jnp.float32),
    )(e_rows, special_tok, w1, b1, w2, b2)


def _ffn_body(seq_ref, wf1_ref, wf2_ref, o_ref, ln_ref):
    j = pl.program_id(1)

    @pl.when(j == 0)
    def _():
        s = seq_ref[...]
        mu = jnp.mean(s, axis=1, keepdims=True)
        var = jnp.mean(s * s, axis=1, keepdims=True) - mu * mu
        ln_ref[...] = ((s - mu) * lax.rsqrt(var + 1e-5)).astype(jnp.bfloat16)
        o_ref[...] = s

    t = jnp.dot(ln_ref[...], wf1_ref[...], preferred_element_type=jnp.float32)
    r = jnp.maximum(t, 0.0).astype(jnp.bfloat16)
    o_ref[...] += jnp.dot(r, wf2_ref[...], preferred_element_type=jnp.float32)


def _ffn_call(seq, wf1, wf2, ti, tj, n=None, base=0):
    if n is None:
        n = seq.shape[0]
    assert n % ti == 0 and base % ti == 0
    boff = base // ti
    return pl.pallas_call(
        _ffn_body,
        grid=(n // ti, (4 * OUT) // tj),
        in_specs=[
            pl.BlockSpec((ti, OUT), lambda i, j: (i + boff, 0)),
            pl.BlockSpec((OUT, tj), lambda i, j: (0, j)),
            pl.BlockSpec((tj, OUT), lambda i, j: (j, 0)),
        ],
        out_specs=pl.BlockSpec((ti, OUT), lambda i, j: (i, 0)),
        out_shape=jax.ShapeDtypeStruct((n, OUT), jnp.float32),
        scratch_shapes=[pltpu.VMEM((ti, OUT), jnp.bfloat16)],
        compiler_params=pltpu.CompilerParams(
            vmem_limit_bytes=63 * 1024 * 1024),
    )(seq, wf1, wf2)


def kernel(x, embeddings, emb_table, special_tok, W1, b1, W2, b2,
           tok_table, Wf1, Wf2):
    x = x.astype(jnp.int32)
    embeddings = embeddings.astype(jnp.int32)

    tok_rows = _gather_tok(tok_table, x)
    e_rows = _gather_emb(emb_table, embeddings)

    wf1, wf2 = _cast_weights(Wf1, Wf2)

    h = _bridge_call(e_rows, special_tok, W1, b1.reshape(1, HID),
                     W2, b2.reshape(1, OUT))

    out_h = _ffn_call(h, wf1, wf2, 1088, 512)
    out_toks = [_ffn_call(tok_rows, wf1, wf2, 1024, 512, n=S, base=b * S)
                for b in range(B)]

    g = out_h[:N_EMB].reshape(B, SEGS * SEG_LEN, OUT)
    sp = out_h[N_EMB:N_EMB + 1]
    rows = [jnp.concatenate(
        [g[b, :SEG_LEN], sp, g[b, SEG_LEN:], sp, out_toks[b]], axis=0)
        for b in range(B)]
    return jnp.stack(rows, axis=0)


# speedup vs baseline: 1.2619x; 1.2619x over previous
import functools

import jax
import jax.numpy as jnp
from jax import lax
from jax.experimental import pallas as pl
from jax.experimental.pallas import tpu as pltpu
from jax.experimental.pallas import tpu_sc as plsc

B = 4
S = 2048
IN_DIM = 1024
HID = 2048
OUT = 2048
SEGS = 2
SEG_LEN = 256

N_EMB = B * SEGS * SEG_LEN
N_TOK = B * S
CAT = SEGS * (SEG_LEN + 1)
N_SEQ = B * (CAT + S)

NC = 2
NS = 16
NW = NC * NS

_sc_mesh = plsc.VectorSubcoreMesh(core_axis_name="c", subcore_axis_name="s")


def _make_sc_gather(n_rows, dim, chunk):
    per_w = n_rows // NW
    n_chunks = per_w // chunk
    assert per_w % chunk == 0 and n_rows % NW == 0 and chunk % 8 == 0

    @functools.partial(
        pl.kernel,
        out_type=jax.ShapeDtypeStruct((n_rows, dim), jnp.float32),
        mesh=_sc_mesh,
        scratch_types=[
            pltpu.VMEM((per_w,), jnp.int32),
            pltpu.VMEM((chunk, dim), jnp.float32),
            pltpu.VMEM((chunk, dim), jnp.float32),
            pltpu.SemaphoreType.DMA,
            pltpu.SemaphoreType.DMA,
            pltpu.SemaphoreType.DMA,
            pltpu.SemaphoreType.DMA,
        ],
    )
    def gather(table_hbm, idx_hbm, out_hbm, idx_v, rows0, rows1,
               sg0, sg1, sw0, sw1):
        wid = lax.axis_index("s") * NC + lax.axis_index("c")
        base = wid * per_w
        pltpu.sync_copy(idx_hbm.at[pl.ds(base, per_w)], idx_v)
        bufs = [(rows0, sg0, sw0), (rows1, sg1, sw1)]
        g = [None] * n_chunks
        w = [None] * n_chunks
        for c in range(n_chunks + 1):
            if c < n_chunks:
                rows, sg, sw = bufs[c % 2]
                if c >= 2:
                    w[c - 2].wait()
                g[c] = pltpu.async_copy(
                    table_hbm.at[idx_v.at[pl.ds(c * chunk, chunk)]], rows, sg)
            if c >= 1:
                rows_p, _, sw_p = bufs[(c - 1) % 2]
                g[c - 1].wait()
                w[c - 1] = pltpu.async_copy(
                    rows_p, out_hbm.at[pl.ds(base + (c - 1) * chunk, chunk)],
                    sw_p)
        for c in range(max(0, n_chunks - 2), n_chunks):
            w[c].wait()

    return gather


_gather_emb = _make_sc_gather(N_EMB, IN_DIM, 16)
_gather_tok = _make_sc_gather(N_TOK, OUT, 16)


_AS_CHUNK = 16


def _copy_rows(src, dst, src_off, dst_off, n_rows, bufs, start_c=0):
    n_chunks = n_rows // _AS_CHUNK
    cw = _AS_CHUNK * OUT
    g = [None] * n_chunks
    w = [None] * n_chunks
    nb = len(bufs)
    for c in range(n_chunks + 1):
        if c < n_chunks:
            rows, sg, sw = bufs[(start_c + c) % nb]
            if c >= nb:
                w[c - nb].wait()
            g[c] = pltpu.async_copy(
                src.at[pl.ds((src_off + c * _AS_CHUNK) * OUT, cw)], rows, sg)
        if c >= 1:
            rows_p, _, sw_p = bufs[(start_c + c - 1) % nb]
            g[c - 1].wait()
            w[c - 1] = pltpu.async_copy(
                rows_p,
                dst.at[pl.ds((dst_off + (c - 1) * _AS_CHUNK) * OUT, cw)],
                sw_p)
    for c in range(max(0, n_chunks - nb), n_chunks):
        w[c].wait()
    return start_c + n_chunks


@functools.partial(
    pl.kernel,
    out_type=jax.ShapeDtypeStruct((N_SEQ * OUT,), jnp.float32),
    mesh=_sc_mesh,
    scratch_types=[
        pltpu.VMEM((_AS_CHUNK * OUT,), jnp.float32),
        pltpu.VMEM((_AS_CHUNK * OUT,), jnp.float32),
        pltpu.VMEM((OUT,), jnp.float32),
        pltpu.SemaphoreType.DMA,
        pltpu.SemaphoreType.DMA,
        pltpu.SemaphoreType.DMA,
        pltpu.SemaphoreType.DMA,
    ],
)
def _assemble(h_hbm, tok_hbm, out_hbm, rows0, rows1, sp_v,
              sg0, sg1, sw0, sw1):
    wid = lax.axis_index("s") * NC + lax.axis_index("c")
    bufs = [(rows0, sg0, sw0), (rows1, sg1, sw1)]
    t0 = wid * 256
    tb = wid // 8
    nxt = _copy_rows(tok_hbm, out_hbm, t0,
                     (CAT + S) * tb + CAT + (t0 - S * tb), 256, bufs)
    g0 = wid * 64
    gb = wid // 8
    gs = (wid % 8) // 4
    gp = (wid % 4) * 64
    _copy_rows(h_hbm, out_hbm, g0,
               (CAT + S) * gb + (SEG_LEN + 1) * gs + gp, 64, bufs,
               start_c=nxt)
    @pl.when(wid < 8)
    def _():
        pltpu.sync_copy(h_hbm.at[pl.ds(N_EMB * OUT, OUT)], sp_v)
        sb = wid // 2
        ss = wid % 2
        pltpu.sync_copy(
            sp_v,
            out_hbm.at[pl.ds(((CAT + S) * sb + (SEG_LEN + 1) * ss + SEG_LEN)
                             * OUT, OUT)])


_CAST_G = 16


def _cast_body(a_ref, b_ref, ao_ref, bo_ref):
    ao_ref[...] = a_ref[...].astype(jnp.bfloat16)
    bo_ref[...] = b_ref[...].astype(jnp.bfloat16)


def _cast_weights(wf1, wf2):
    shapes = [wf1.shape, wf2.shape]
    blocks = [(s[0] // _CAST_G, s[1]) for s in shapes]
    return pl.pallas_call(
        _cast_body,
        grid=(_CAST_G,),
        in_specs=[pl.BlockSpec(blk, lambda i: (i, 0)) for blk in blocks],
        out_specs=[pl.BlockSpec(blk, lambda i: (i, 0)) for blk in blocks],
        out_shape=[jax.ShapeDtypeStruct(s, jnp.bfloat16) for s in shapes],
    )(wf1, wf2)


_BR_ROWS = N_EMB + 128
_BR_TI = 128
_BR_NB = _BR_ROWS // _BR_TI


def _bridge_body(e_ref, sp_ref, w1_ref, b1_ref, w2_ref, b2_ref, o_ref):
    i = pl.program_id(0)
    e = e_ref[...]
    sp = jnp.broadcast_to(sp_ref[...], e.shape)
    e = jnp.where(i == _BR_NB - 1, sp, e)
    h = jnp.dot(e, w1_ref[...], preferred_element_type=jnp.float32)
    h = jax.nn.gelu(h + b1_ref[...])
    o_ref[...] = (
        jnp.dot(h, w2_ref[...], preferred_element_type=jnp.float32)
        + b2_ref[...]
    )


def _bridge_call(e_rows, special_tok, w1, b1, w2, b2):
    nb = _BR_NB
    return pl.pallas_call(
        _bridge_body,
        grid=(nb,),
        in_specs=[
            pl.BlockSpec((_BR_TI, IN_DIM),
                         lambda i: (jnp.minimum(i, nb - 2), 0)),
            pl.BlockSpec((1, IN_DIM), lambda i: (0, 0)),
            pl.BlockSpec((IN_DIM, HID), lambda i: (0, 0)),
            pl.BlockSpec((1, HID), lambda i: (0, 0)),
            pl.BlockSpec((HID, OUT), lambda i: (0, 0)),
            pl.BlockSpec((1, OUT), lambda i: (0, 0)),
        ],
        out_specs=pl.BlockSpec((_BR_TI, OUT), lambda i: (i, 0)),
        out_shape=jax.ShapeDtypeStruct((_BR_ROWS, OUT), jnp.float32),
    )(e_rows, special_tok, w1, b1, w2, b2)


def _ffn_body(seq_ref, wf1_ref, wf2_ref, o_ref, ln_ref):
    j = pl.program_id(1)

    @pl.when(j == 0)
    def _():
        s = seq_ref[...]
        mu = jnp.mean(s, axis=1, keepdims=True)
        var = jnp.mean(s * s, axis=1, keepdims=True) - mu * mu
        ln_ref[...] = ((s - mu) * lax.rsqrt(var + 1e-5)).astype(jnp.bfloat16)
        o_ref[...] = s

    t = jnp.dot(ln_ref[...], wf1_ref[...], preferred_element_type=jnp.float32)
    r = jnp.maximum(t, 0.0).astype(jnp.bfloat16)
    o_ref[...] += jnp.dot(r, wf2_ref[...], preferred_element_type=jnp.float32)


def _ffn_call(seq, wf1, wf2, ti, tj, n=None, base=0):
    if n is None:
        n = seq.shape[0]
    assert n % ti == 0 and base % ti == 0
    boff = base // ti
    return pl.pallas_call(
        _ffn_body,
        grid=(n // ti, (4 * OUT) // tj),
        in_specs=[
            pl.BlockSpec((ti, OUT), lambda i, j: (i + boff, 0)),
            pl.BlockSpec((OUT, tj), lambda i, j: (0, j)),
            pl.BlockSpec((tj, OUT), lambda i, j: (j, 0)),
        ],
        out_specs=pl.BlockSpec((ti, OUT), lambda i, j: (i, 0)),
        out_shape=jax.ShapeDtypeStruct((n, OUT), jnp.float32),
        scratch_shapes=[pltpu.VMEM((ti, OUT), jnp.bfloat16)],
        compiler_params=pltpu.CompilerParams(
            vmem_limit_bytes=63 * 1024 * 1024),
    )(seq, wf1, wf2)


def kernel(x, embeddings, emb_table, special_tok, W1, b1, W2, b2,
           tok_table, Wf1, Wf2):
    x = x.astype(jnp.int32)
    embeddings = embeddings.astype(jnp.int32)

    tok_rows = _gather_tok(tok_table, x)
    e_rows = _gather_emb(emb_table, embeddings)

    wf1, wf2 = _cast_weights(Wf1, Wf2)

    h = _bridge_call(e_rows, special_tok, W1, b1.reshape(1, HID),
                     W2, b2.reshape(1, OUT))

    out_h = _ffn_call(h, wf1, wf2, 1088, 512)
    out_tok = _ffn_call(tok_rows, wf1, wf2, 1024, 512)

    g = out_h[:N_EMB].reshape(B, SEGS * SEG_LEN, OUT)
    sp = jnp.broadcast_to(out_h[N_EMB:N_EMB + 1].reshape(1, 1, OUT),
                          (B, 1, OUT))
    t = out_tok.reshape(B, S, OUT)
    return jnp.concatenate(
        [g[:, :SEG_LEN], sp, g[:, SEG_LEN:], sp, t], axis=1)


# speedup vs baseline: 1.2686x; 1.0052x over previous
import functools

import jax
import jax.numpy as jnp
from jax import lax
from jax.experimental import pallas as pl
from jax.experimental.pallas import tpu as pltpu
from jax.experimental.pallas import tpu_sc as plsc

B = 4
S = 2048
IN_DIM = 1024
HID = 2048
OUT = 2048
SEGS = 2
SEG_LEN = 256

N_EMB = B * SEGS * SEG_LEN
N_TOK = B * S
CAT = SEGS * (SEG_LEN + 1)
N_SEQ = B * (CAT + S)

NC = 2
NS = 16
NW = NC * NS

_sc_mesh = plsc.VectorSubcoreMesh(core_axis_name="c", subcore_axis_name="s")


def _make_sc_gather(n_rows, dim, chunk):
    per_w = n_rows // NW
    n_chunks = per_w // chunk
    assert per_w % chunk == 0 and n_rows % NW == 0 and chunk % 8 == 0

    @functools.partial(
        pl.kernel,
        out_type=jax.ShapeDtypeStruct((n_rows, dim), jnp.float32),
        mesh=_sc_mesh,
        scratch_types=[
            pltpu.VMEM((per_w,), jnp.int32),
            pltpu.VMEM((chunk, dim), jnp.float32),
            pltpu.VMEM((chunk, dim), jnp.float32),
            pltpu.SemaphoreType.DMA,
            pltpu.SemaphoreType.DMA,
            pltpu.SemaphoreType.DMA,
            pltpu.SemaphoreType.DMA,
        ],
    )
    def gather(table_hbm, idx_hbm, out_hbm, idx_v, rows0, rows1,
               sg0, sg1, sw0, sw1):
        wid = lax.axis_index("s") * NC + lax.axis_index("c")
        base = wid * per_w
        pltpu.sync_copy(idx_hbm.at[pl.ds(base, per_w)], idx_v)
        bufs = [(rows0, sg0, sw0), (rows1, sg1, sw1)]
        g = [None] * n_chunks
        w = [None] * n_chunks
        for c in range(n_chunks + 1):
            if c < n_chunks:
                rows, sg, sw = bufs[c % 2]
                if c >= 2:
                    w[c - 2].wait()
                g[c] = pltpu.async_copy(
                    table_hbm.at[idx_v.at[pl.ds(c * chunk, chunk)]], rows, sg)
            if c >= 1:
                rows_p, _, sw_p = bufs[(c - 1) % 2]
                g[c - 1].wait()
                w[c - 1] = pltpu.async_copy(
                    rows_p, out_hbm.at[pl.ds(base + (c - 1) * chunk, chunk)],
                    sw_p)
        for c in range(max(0, n_chunks - 2), n_chunks):
            w[c].wait()

    return gather


_gather_emb = _make_sc_gather(N_EMB, IN_DIM, 16)
_gather_tok = _make_sc_gather(N_TOK, OUT, 16)


_AS_CHUNK = 16


def _copy_rows(src, dst, src_off, dst_off, n_rows, bufs, start_c=0):
    n_chunks = n_rows // _AS_CHUNK
    cw = _AS_CHUNK * OUT
    g = [None] * n_chunks
    w = [None] * n_chunks
    nb = len(bufs)
    for c in range(n_chunks + 1):
        if c < n_chunks:
            rows, sg, sw = bufs[(start_c + c) % nb]
            if c >= nb:
                w[c - nb].wait()
            g[c] = pltpu.async_copy(
                src.at[pl.ds((src_off + c * _AS_CHUNK) * OUT, cw)], rows, sg)
        if c >= 1:
            rows_p, _, sw_p = bufs[(start_c + c - 1) % nb]
            g[c - 1].wait()
            w[c - 1] = pltpu.async_copy(
                rows_p,
                dst.at[pl.ds((dst_off + (c - 1) * _AS_CHUNK) * OUT, cw)],
                sw_p)
    for c in range(max(0, n_chunks - nb), n_chunks):
        w[c].wait()
    return start_c + n_chunks


@functools.partial(
    pl.kernel,
    out_type=jax.ShapeDtypeStruct((N_SEQ * OUT,), jnp.float32),
    mesh=_sc_mesh,
    scratch_types=[
        pltpu.VMEM((_AS_CHUNK * OUT,), jnp.float32),
        pltpu.VMEM((_AS_CHUNK * OUT,), jnp.float32),
        pltpu.VMEM((OUT,), jnp.float32),
        pltpu.SemaphoreType.DMA,
        pltpu.SemaphoreType.DMA,
        pltpu.SemaphoreType.DMA,
        pltpu.SemaphoreType.DMA,
    ],
)
def _assemble(h_hbm, tok_hbm, out_hbm, rows0, rows1, sp_v,
              sg0, sg1, sw0, sw1):
    wid = lax.axis_index("s") * NC + lax.axis_index("c")
    bufs = [(rows0, sg0, sw0), (rows1, sg1, sw1)]
    t0 = wid * 256
    tb = wid // 8
    nxt = _copy_rows(tok_hbm, out_hbm, t0,
                     (CAT + S) * tb + CAT + (t0 - S * tb), 256, bufs)
    g0 = wid * 64
    gb = wid // 8
    gs = (wid % 8) // 4
    gp = (wid % 4) * 64
    _copy_rows(h_hbm, out_hbm, g0,
               (CAT + S) * gb + (SEG_LEN + 1) * gs + gp, 64, bufs,
               start_c=nxt)
    @pl.when(wid < 8)
    def _():
        pltpu.sync_copy(h_hbm.at[pl.ds(N_EMB * OUT, OUT)], sp_v)
        sb = wid // 2
        ss = wid % 2
        pltpu.sync_copy(
            sp_v,
            out_hbm.at[pl.ds(((CAT + S) * sb + (SEG_LEN + 1) * ss + SEG_LEN)
                             * OUT, OUT)])


_CAST_G = 16


def _cast_body(a_ref, b_ref, ao_ref, bo_ref):
    ao_ref[...] = a_ref[...].astype(jnp.bfloat16)
    bo_ref[...] = b_ref[...].astype(jnp.bfloat16)


def _cast_weights(wf1, wf2):
    shapes = [wf1.shape, wf2.shape]
    blocks = [(s[0] // _CAST_G, s[1]) for s in shapes]
    return pl.pallas_call(
        _cast_body,
        grid=(_CAST_G,),
        in_specs=[pl.BlockSpec(blk, lambda i: (i, 0)) for blk in blocks],
        out_specs=[pl.BlockSpec(blk, lambda i: (i, 0)) for blk in blocks],
        out_shape=[jax.ShapeDtypeStruct(s, jnp.bfloat16) for s in shapes],
    )(wf1, wf2)


_BR_ROWS = N_EMB + 128
_BR_TI = 128
_BR_NB = _BR_ROWS // _BR_TI


def _bridge_body(e_ref, sp_ref, w1_ref, b1_ref, w2_ref, b2_ref, o_ref):
    i = pl.program_id(0)
    e = e_ref[...]
    sp = jnp.broadcast_to(sp_ref[...], e.shape)
    e = jnp.where(i == _BR_NB - 1, sp, e)
    h = jnp.dot(e, w1_ref[...], preferred_element_type=jnp.float32)
    h = jax.nn.gelu(h + b1_ref[...])
    o_ref[...] = (
        jnp.dot(h, w2_ref[...], preferred_element_type=jnp.float32)
        + b2_ref[...]
    )


def _bridge_call(e_rows, special_tok, w1, b1, w2, b2):
    nb = _BR_NB
    return pl.pallas_call(
        _bridge_body,
        grid=(nb,),
        in_specs=[
            pl.BlockSpec((_BR_TI, IN_DIM),
                         lambda i: (jnp.minimum(i, nb - 2), 0)),
            pl.BlockSpec((1, IN_DIM), lambda i: (0, 0)),
            pl.BlockSpec((IN_DIM, HID), lambda i: (0, 0)),
            pl.BlockSpec((1, HID), lambda i: (0, 0)),
            pl.BlockSpec((HID, OUT), lambda i: (0, 0)),
            pl.BlockSpec((1, OUT), lambda i: (0, 0)),
        ],
        out_specs=pl.BlockSpec((_BR_TI, OUT), lambda i: (i, 0)),
        out_shape=jax.ShapeDtypeStruct((_BR_ROWS, OUT), jnp.float32),
    )(e_rows, special_tok, w1, b1, w2, b2)


def _ffn_body(seq_ref, wf1_ref, wf2_ref, o_ref, ln_ref):
    j = pl.program_id(1)

    @pl.when(j == 0)
    def _():
        s = seq_ref[...]
        mu = jnp.mean(s, axis=1, keepdims=True)
        var = jnp.mean((s - mu) ** 2, axis=1, keepdims=True)
        ln_ref[...] = ((s - mu) * lax.rsqrt(var + 1e-5)).astype(jnp.bfloat16)
        o_ref[...] = s

    t = jnp.dot(ln_ref[...], wf1_ref[...], preferred_element_type=jnp.float32)
    r = jnp.maximum(t, 0.0).astype(jnp.bfloat16)
    o_ref[...] += jnp.dot(r, wf2_ref[...], preferred_element_type=jnp.float32)


def _ffn_call(seq, wf1, wf2, ti, tj, n=None, base=0):
    if n is None:
        n = seq.shape[0]
    assert n % ti == 0 and base % ti == 0
    boff = base // ti
    return pl.pallas_call(
        _ffn_body,
        grid=(n // ti, (4 * OUT) // tj),
        in_specs=[
            pl.BlockSpec((ti, OUT), lambda i, j: (i + boff, 0)),
            pl.BlockSpec((OUT, tj), lambda i, j: (0, j)),
            pl.BlockSpec((tj, OUT), lambda i, j: (j, 0)),
        ],
        out_specs=pl.BlockSpec((ti, OUT), lambda i, j: (i, 0)),
        out_shape=jax.ShapeDtypeStruct((n, OUT), jnp.float32),
        scratch_shapes=[pltpu.VMEM((ti, OUT), jnp.bfloat16)],
        compiler_params=pltpu.CompilerParams(
            vmem_limit_bytes=63 * 1024 * 1024),
    )(seq, wf1, wf2)


def kernel(x, embeddings, emb_table, special_tok, W1, b1, W2, b2,
           tok_table, Wf1, Wf2):
    x = x.astype(jnp.int32)
    embeddings = embeddings.astype(jnp.int32)

    tok_rows = _gather_tok(tok_table, x)
    e_rows = _gather_emb(emb_table, embeddings)

    wf1, wf2 = _cast_weights(Wf1, Wf2)

    h = _bridge_call(e_rows, special_tok, W1, b1.reshape(1, HID),
                     W2, b2.reshape(1, OUT))

    out_h = _ffn_call(h, wf1, wf2, 1088, 512)
    out_tok = _ffn_call(tok_rows, wf1, wf2, 1024, 512)

    g = out_h[:N_EMB].reshape(B, SEGS * SEG_LEN, OUT)
    sp = jnp.broadcast_to(out_h[N_EMB:N_EMB + 1].reshape(1, 1, OUT),
                          (B, 1, OUT))
    t = out_tok.reshape(B, S, OUT)
    return jnp.concatenate(
        [g[:, :SEG_LEN], sp, g[:, SEG_LEN:], sp, t], axis=1)
